# Initial kernel scaffold; baseline (speedup 1.0000x reference)
import jax
import jax.numpy as jnp
from jax.experimental import pallas as pl


def kernel(given4rank1_stimulus_set, table):
    raise NotImplementedError("write your pallas kernel here")


# speedup vs baseline: 7.8778x; 7.8778x over previous
import functools

import jax
import jax.numpy as jnp
from jax import lax
from jax.experimental import pallas as pl
from jax.experimental.pallas import tpu as pltpu
from jax.experimental.pallas import tpu_sc as plsc

N_STIM = 21
N_DIM = 3
BATCH = 16384
NC, NS = 2, 16
NW = NC * NS
ROWS_PER_W = BATCH // NW
GROUPS = ROWS_PER_W // 16
NPAIR = N_STIM * N_STIM
PAIR_VECS = (NPAIR + 15) // 16
SIM_PAD = PAIR_VECS * 16


@functools.partial(
    pl.kernel,
    mesh=plsc.VectorSubcoreMesh(core_axis_name="c", subcore_axis_name="s"),
    out_type=jax.ShapeDtypeStruct((BATCH, 4), jnp.float32),
    compiler_params=pltpu.CompilerParams(
        needs_layout_passes=False, use_tc_tiling_on_sc=False),
    scratch_types=[
        pltpu.VMEM((ROWS_PER_W, 5), jnp.int32),
        pltpu.VMEM((64,), jnp.float32),
        pltpu.VMEM((SIM_PAD,), jnp.float32),
        pltpu.VMEM((ROWS_PER_W, 4), jnp.float32),
    ],
)
def _rank_sc(stim_hbm, tab_hbm, out_hbm, idx_v, tab_v, sim_v, out_v):
    wid = lax.axis_index("s") * NC + lax.axis_index("c")
    base = wid * ROWS_PER_W
    pltpu.sync_copy(stim_hbm.at[pl.ds(base, ROWS_PER_W)], idx_v)
    pltpu.sync_copy(tab_hbm, tab_v)

    lanes = lax.iota(jnp.int32, 16)

    def build_sim(v, carry):
        p = jnp.minimum(v * 16 + lanes, NPAIR - 1)
        i = p // N_STIM
        j = p - i * N_STIM
        d2 = jnp.zeros((16,), jnp.float32)
        for d in range(N_DIM):
            xi = plsc.load_gather(tab_v, [i * N_DIM + d])
            xj = plsc.load_gather(tab_v, [j * N_DIM + d])
            diff = xi - xj
            d2 = d2 + diff * diff
        bits = lax.bitcast_convert_type(d2, jnp.int32)
        y = lax.bitcast_convert_type(
            jnp.int32(0x5F3759DF) - lax.shift_right_logical(bits, 1),
            jnp.float32)
        for _ in range(3):
            y = y * (1.5 - 0.5 * d2 * y * y)
        dist = d2 * y
        plsc.store_scatter(sim_v, [p], jnp.exp(-10.0 * dist))
        return carry

    lax.fori_loop(0, PAIR_VECS, build_sim, None)

    def do_group(g, carry):
        rows = g * 16 + lanes
        q = plsc.load_gather(idx_v, [rows, jnp.zeros((16,), jnp.int32)])
        qbase = q * N_STIM
        s = []
        for k in range(4):
            r = plsc.load_gather(idx_v, [rows, jnp.full((16,), k + 1, jnp.int32)])
            s.append(plsc.load_gather(sim_v, [qbase + r]))
        inv = 1.0 / (s[0] + s[1] + s[2] + s[3])
        for k in range(4):
            plsc.store_scatter(out_v, [rows, jnp.full((16,), k, jnp.int32)],
                               s[k] * inv)
        return carry

    lax.fori_loop(0, GROUPS, do_group, None)

    pltpu.sync_copy(out_v, out_hbm.at[pl.ds(base, ROWS_PER_W)])


def kernel(given4rank1_stimulus_set, table):
    tab_flat = jnp.pad(jnp.reshape(table, (-1,)), (0, 64 - N_STIM * N_DIM))
    return _rank_sc(given4rank1_stimulus_set, tab_flat)


# speedup vs baseline: 16.5223x; 2.0973x over previous
import functools

import jax
import jax.numpy as jnp
from jax import lax
from jax.experimental import pallas as pl
from jax.experimental.pallas import tpu as pltpu
from jax.experimental.pallas import tpu_sc as plsc

N_STIM = 21
N_DIM = 3
BATCH = 16384
NC, NS = 2, 16
NW = NC * NS
ROWS_PER_W = BATCH // NW
GROUPS = ROWS_PER_W // 16
NPAIR = N_STIM * N_STIM
PAIR_VECS = (NPAIR + 15) // 16
SIM_PAD = PAIR_VECS * 16


@functools.partial(
    pl.kernel,
    mesh=plsc.VectorSubcoreMesh(core_axis_name="c", subcore_axis_name="s"),
    out_type=jax.ShapeDtypeStruct((BATCH * 4,), jnp.float32),
    compiler_params=pltpu.CompilerParams(
        needs_layout_passes=False, use_tc_tiling_on_sc=False),
    scratch_types=[
        pltpu.VMEM((5 * ROWS_PER_W,), jnp.int32),
        pltpu.VMEM((64,), jnp.float32),
        pltpu.VMEM((SIM_PAD,), jnp.float32),
        pltpu.VMEM((4 * ROWS_PER_W,), jnp.float32),
    ],
)
def _rank_sc(stim_hbm, tab_hbm, out_hbm, idx_v, tab_v, sim_v, out_v):
    wid = lax.axis_index("s") * NC + lax.axis_index("c")
    base = wid * ROWS_PER_W
    for k in range(5):
        pltpu.sync_copy(stim_hbm.at[pl.ds(k * BATCH + base, ROWS_PER_W)],
                        idx_v.at[pl.ds(k * ROWS_PER_W, ROWS_PER_W)])
    pltpu.sync_copy(tab_hbm, tab_v)

    lanes = lax.iota(jnp.int32, 16)

    def build_sim(v, carry):
        p = jnp.minimum(v * 16 + lanes, NPAIR - 1)
        i = p // N_STIM
        j = p - i * N_STIM
        d2 = jnp.zeros((16,), jnp.float32)
        for d in range(N_DIM):
            xi = plsc.load_gather(tab_v, [i * N_DIM + d])
            xj = plsc.load_gather(tab_v, [j * N_DIM + d])
            diff = xi - xj
            d2 = d2 + diff * diff
        bits = lax.bitcast_convert_type(d2, jnp.int32)
        y = lax.bitcast_convert_type(
            jnp.int32(0x5F3759DF) - lax.shift_right_logical(bits, 1),
            jnp.float32)
        for _ in range(3):
            y = y * (1.5 - 0.5 * d2 * y * y)
        dist = d2 * y
        plsc.store_scatter(sim_v, [p], jnp.exp(-10.0 * dist))
        return carry

    lax.fori_loop(0, PAIR_VECS, build_sim, None)

    def do_group(g, carry):
        off = g * 16
        q = idx_v[pl.ds(off, 16)]
        qbase = q * N_STIM
        s = []
        for k in range(4):
            r = idx_v[pl.ds((k + 1) * ROWS_PER_W + off, 16)]
            s.append(plsc.load_gather(sim_v, [qbase + r]))
        inv = 1.0 / (s[0] + s[1] + s[2] + s[3])
        obase = (g // 8) * 512 + (g % 8) * 16
        for k in range(4):
            out_v[pl.ds(obase + k * 128, 16)] = s[k] * inv
        return carry

    lax.fori_loop(0, GROUPS, do_group, None)

    pltpu.sync_copy(out_v, out_hbm.at[pl.ds(wid * 4 * ROWS_PER_W,
                                            4 * ROWS_PER_W)])


def kernel(given4rank1_stimulus_set, table):
    stim_cm = jnp.reshape(jnp.transpose(given4rank1_stimulus_set), (5 * BATCH,))
    tab_flat = jnp.pad(jnp.reshape(table, (-1,)), (0, 64 - N_STIM * N_DIM))
    out_flat = _rank_sc(stim_cm, tab_flat)
    out3 = jnp.reshape(out_flat, (BATCH // 128, 4, 128))
    return jnp.reshape(jnp.transpose(out3, (0, 2, 1)), (BATCH, 4))


# speedup vs baseline: 18.2195x; 1.1027x over previous
import functools

import jax
import jax.numpy as jnp
from jax import lax
from jax.experimental import pallas as pl
from jax.experimental.pallas import tpu as pltpu
from jax.experimental.pallas import tpu_sc as plsc

N_STIM = 21
N_DIM = 3
BATCH = 16384
NC, NS = 2, 16
NW = NC * NS
ROWS_PER_W = BATCH // NW
GROUPS = ROWS_PER_W // 16
NPAIR = N_STIM * N_STIM
PAIR_VECS = (NPAIR + 15) // 16
SIM_PAD = PAIR_VECS * 16


@functools.partial(
    pl.kernel,
    mesh=plsc.VectorSubcoreMesh(core_axis_name="c", subcore_axis_name="s"),
    out_type=jax.ShapeDtypeStruct((BATCH * 4,), jnp.float32),
    compiler_params=pltpu.CompilerParams(
        needs_layout_passes=False, use_tc_tiling_on_sc=False),
    scratch_types=[
        pltpu.VMEM((5, ROWS_PER_W), jnp.int32),
        pltpu.VMEM((64,), jnp.float32),
        pltpu.VMEM((SIM_PAD,), jnp.float32),
        pltpu.VMEM((4 * ROWS_PER_W,), jnp.float32),
        pltpu.SemaphoreType.DMA,
    ],
)
def _rank_sc(stim_hbm, tab_hbm, out_hbm, idx_v, tab_v, sim_v, out_v, sem):
    wid = lax.axis_index("s") * NC + lax.axis_index("c")
    base = wid * ROWS_PER_W
    idx_dma = pltpu.async_copy(
        stim_hbm.at[:, pl.ds(base, ROWS_PER_W)], idx_v, sem)
    pltpu.sync_copy(tab_hbm, tab_v)

    lanes = lax.iota(jnp.int32, 16)

    def build_sim(v, carry):
        p = jnp.minimum(v * 16 + lanes, NPAIR - 1)
        i = p // N_STIM
        j = p - i * N_STIM
        d2 = jnp.zeros((16,), jnp.float32)
        for d in range(N_DIM):
            xi = plsc.load_gather(tab_v, [i * N_DIM + d])
            xj = plsc.load_gather(tab_v, [j * N_DIM + d])
            diff = xi - xj
            d2 = d2 + diff * diff
        bits = lax.bitcast_convert_type(d2, jnp.int32)
        y = lax.bitcast_convert_type(
            jnp.int32(0x5F3759DF) - lax.shift_right_logical(bits, 1),
            jnp.float32)
        for _ in range(3):
            y = y * (1.5 - 0.5 * d2 * y * y)
        dist = d2 * y
        plsc.store_scatter(sim_v, [p], jnp.exp(-10.0 * dist))
        return carry

    lax.fori_loop(0, PAIR_VECS, build_sim, None)
    idx_dma.wait()

    def do_group(g, carry):
        off = g * 16
        q = idx_v[0, pl.ds(off, 16)]
        qbase = q * N_STIM
        s = []
        for k in range(4):
            r = idx_v[k + 1, pl.ds(off, 16)]
            s.append(plsc.load_gather(sim_v, [qbase + r]))
        inv = 1.0 / (s[0] + s[1] + s[2] + s[3])
        obase = (g // 8) * 512 + (g % 8) * 16
        for k in range(4):
            out_v[pl.ds(obase + k * 128, 16)] = s[k] * inv
        return carry

    lax.fori_loop(0, GROUPS, do_group, None)

    pltpu.sync_copy(out_v, out_hbm.at[pl.ds(wid * 4 * ROWS_PER_W,
                                            4 * ROWS_PER_W)])


def kernel(given4rank1_stimulus_set, table):
    stim_cm = jnp.transpose(given4rank1_stimulus_set)
    tab_flat = jnp.pad(jnp.reshape(table, (-1,)), (0, 64 - N_STIM * N_DIM))
    out_flat = _rank_sc(stim_cm, tab_flat)
    out3 = jnp.reshape(out_flat, (BATCH // 128, 4, 128))
    return jnp.reshape(jnp.transpose(out3, (0, 2, 1)), (BATCH, 4))
